# Initial kernel scaffold; baseline (speedup 1.0000x reference)
import jax
import jax.numpy as jnp
from jax.experimental import pallas as pl


def kernel(query, reference_points, input_flatten, input_spatial_shapes, Wv, bv, Ws, bs_off, Wa, ba, Wo, bo):
    raise NotImplementedError("write your pallas kernel here")


# speedup vs baseline: 2.3911x; 2.3911x over previous
import jax
import jax.numpy as jnp
from jax import lax
from jax.experimental import pallas as pl
from jax.experimental.pallas import tpu as pltpu
from jax.experimental.pallas import tpu_sc as plsc

D_MODEL = 256
N_HEADS = 8
N_LEVELS = 3
N_POINTS = 2
D_HEAD = 32
BS, NQ, FL = 2, 900, 3
H = W = 100
HW = H * W
NQF = NQ * FL
NHLP = N_HEADS * N_LEVELS * N_POINTS
R_ROWS = BS * NQF * N_HEADS
NCON = N_LEVELS * N_POINTS * 4

NW = 32
R_PAD = 45056
ROWS_W = R_PAD // NW
CHUNK = 128
NCHUNK = ROWS_W // CHUNK
IDX_ROWS_W = CHUNK * NCON // 128


def _value_kernel(x_ref, wv_ref, bv_ref, out_ref):
    v = jnp.dot(x_ref[0], wv_ref[...], preferred_element_type=jnp.float32)
    out_ref[0] = v + bv_ref[...][None, :]


def _value_table(input_flatten, Wv, bv):
    thw = 2000
    out = pl.pallas_call(
        _value_kernel,
        grid=(BS, HW // thw),
        in_specs=[
            pl.BlockSpec((1, thw, D_MODEL), lambda b, i: (b, i, 0)),
            pl.BlockSpec((D_MODEL, D_MODEL), lambda b, i: (0, 0)),
            pl.BlockSpec((D_MODEL,), lambda b, i: (0,)),
        ],
        out_specs=pl.BlockSpec((1, thw, D_MODEL), lambda b, i: (b, i, 0)),
        out_shape=jax.ShapeDtypeStruct((BS, HW, D_MODEL), jnp.float32),
    )(input_flatten, Wv, bv)
    return out.reshape(BS * HW * N_HEADS, D_HEAD)


def _prep_kernel(qt_ref, rpt_ref, wst_ref, bst_ref, wat_ref, bat_ref,
                 idx_ref, w_ref):
    b = pl.program_id(0)
    qt = qt_ref[0]
    off = jnp.dot(wst_ref[...], qt, preferred_element_type=jnp.float32)
    off = off + bst_ref[...]
    a = jnp.dot(wat_ref[...], qt, preferred_element_type=jnp.float32)
    a = a + bat_ref[...]
    a3 = a.reshape(N_HEADS, N_LEVELS * N_POINTS, NQF)
    m = jnp.max(a3, axis=1, keepdims=True)
    e = jnp.exp(a3 - m)
    aw = (e / jnp.sum(e, axis=1, keepdims=True)).reshape(NHLP, NQF)

    rpx = rpt_ref[0, 0:1]
    rpy = rpt_ref[0, 1:2]
    x = (rpx + off[0:NHLP] / 100.0) * 100.0 - 0.5
    y = (rpy + off[NHLP:2 * NHLP] / 100.0) * 100.0 - 0.5
    x0 = jnp.floor(x)
    y0 = jnp.floor(y)
    wx1 = x - x0
    wx0 = 1.0 - wx1
    wy1 = y - y0
    wy0 = 1.0 - wy1

    h_arr = lax.broadcasted_iota(jnp.int32, (NHLP, NQF), 0) // (N_LEVELS * N_POINTS)
    for c, (cy, wy, cx, wx) in enumerate(
            ((0, wy0, 0, wx0), (0, wy0, 1, wx1), (1, wy1, 0, wx0), (1, wy1, 1, wx1))):
        yy = y0 + cy
        xx = x0 + cx
        yi = jnp.clip(yy.astype(jnp.int32), 0, H - 1)
        xi = jnp.clip(xx.astype(jnp.int32), 0, W - 1)
        valid = (yy >= 0) & (yy <= float(H - 1)) & (xx >= 0) & (xx <= float(W - 1))
        idx_ref[0, c] = (b * HW + yi * W + xi) * N_HEADS + h_arr
        w_ref[0, c] = wy * wx * valid.astype(jnp.float32) * aw


def _prep(q_t, rp_t, WsT, bsT, WaT, baT):
    idx, w = pl.pallas_call(
        _prep_kernel,
        grid=(BS,),
        in_specs=[
            pl.BlockSpec((1, D_MODEL, NQF), lambda b: (b, 0, 0)),
            pl.BlockSpec((1, 2, NQF), lambda b: (b, 0, 0)),
            pl.BlockSpec((2 * NHLP, D_MODEL), lambda b: (0, 0)),
            pl.BlockSpec((2 * NHLP, 1), lambda b: (0, 0)),
            pl.BlockSpec((NHLP, D_MODEL), lambda b: (0, 0)),
            pl.BlockSpec((NHLP, 1), lambda b: (0, 0)),
        ],
        out_specs=[
            pl.BlockSpec((1, 4, NHLP, NQF), lambda b: (b, 0, 0, 0)),
            pl.BlockSpec((1, 4, NHLP, NQF), lambda b: (b, 0, 0, 0)),
        ],
        out_shape=[
            jax.ShapeDtypeStruct((BS, 4, NHLP, NQF), jnp.int32),
            jax.ShapeDtypeStruct((BS, 4, NHLP, NQF), jnp.float32),
        ],
    )(q_t, rp_t, WsT, bsT, WaT, baT)
    return idx, w


def _sc_gather_kernel(val_hbm, idx_hbm, w_hbm, out_hbm, idx_v, w_v, rows_v,
                      out_v, sem):
    wid = lax.axis_index("s") * 2 + lax.axis_index("c")

    def chunk_body(c, _):
        base = pl.multiple_of(wid * ROWS_W + c * CHUNK, 128)
        irow = pl.multiple_of(base * NCON // 128, 8)
        pltpu.sync_copy(idx_hbm.at[pl.ds(irow, IDX_ROWS_W)], idx_v)
        pltpu.sync_copy(w_hbm.at[pl.ds(base * NCON, CHUNK * NCON)], w_v)
        copies = []
        for j in range(IDX_ROWS_W):
            copies.append(
                pltpu.async_copy(val_hbm.at[idx_v.at[j]],
                                 rows_v.at[pl.ds(j * 128, 128)], sem))
        for cp in copies:
            cp.wait()

        def row_body(j, _):
            s = j * NCON
            wv0 = w_v[pl.ds(s, 16)]
            wv1 = w_v[pl.ds(s + 8, 16)]
            acc0 = jnp.zeros((16,), jnp.float32)
            acc1 = jnp.zeros((16,), jnp.float32)
            for k in range(NCON):
                wk = wv0[k] if k < 16 else wv1[k - 8]
                acc0 = acc0 + wk * rows_v[s + k, pl.ds(0, 16)]
                acc1 = acc1 + wk * rows_v[s + k, pl.ds(16, 16)]
            out_v[j, pl.ds(0, 16)] = acc0
            out_v[j, pl.ds(16, 16)] = acc1
            return 0

        lax.fori_loop(0, CHUNK, row_body, 0)
        pltpu.sync_copy(out_v, out_hbm.at[pl.ds(base, CHUNK)])
        return 0

    lax.fori_loop(0, NCHUNK, chunk_body, 0)


def _sc_gather(val_t, idx_pad2d, w_pad):
    mesh = plsc.VectorSubcoreMesh(core_axis_name="c", subcore_axis_name="s")
    run = pl.kernel(
        _sc_gather_kernel,
        mesh=mesh,
        compiler_params=pltpu.CompilerParams(use_tc_tiling_on_sc=False),
        out_type=jax.ShapeDtypeStruct((R_PAD, D_HEAD), jnp.float32),
        scratch_types=[
            pltpu.VMEM((IDX_ROWS_W, 128), jnp.int32),
            pltpu.VMEM((CHUNK * NCON,), jnp.float32),
            pltpu.VMEM((CHUNK * NCON, D_HEAD), jnp.float32),
            pltpu.VMEM((CHUNK, D_HEAD), jnp.float32),
            pltpu.SemaphoreType.DMA,
        ],
    )
    return run(val_t, idx_pad2d, w_pad)


def _final_kernel(s_ref, wo_ref, bo_ref, out_ref):
    out_ref[0] = (jnp.dot(s_ref[0], wo_ref[...], preferred_element_type=jnp.float32)
                  + bo_ref[...][None, :])


def _final(out_rows, Wo, bo):
    out = pl.pallas_call(
        _final_kernel,
        grid=(BS,),
        in_specs=[
            pl.BlockSpec((1, NQF, D_MODEL), lambda b: (b, 0, 0)),
            pl.BlockSpec((D_MODEL, D_MODEL), lambda b: (0, 0)),
            pl.BlockSpec((D_MODEL,), lambda b: (0,)),
        ],
        out_specs=pl.BlockSpec((1, NQF, D_MODEL), lambda b: (b, 0, 0)),
        out_shape=jax.ShapeDtypeStruct((BS, NQF, D_MODEL), jnp.float32),
    )(out_rows, Wo, bo)
    return out.reshape(BS, NQ, FL, D_MODEL)


@jax.jit
def _run(query, reference_points, input_flatten, Wv, bv, Ws, bs_off, Wa, ba,
         Wo, bo):
    q_t = query.reshape(BS, NQF, D_MODEL).transpose(0, 2, 1)
    rp_t = reference_points.reshape(BS, NQF, 2).transpose(0, 2, 1)
    WsT = (Ws.T.reshape(N_HEADS, N_LEVELS, N_POINTS, 2, D_MODEL)
           .transpose(3, 0, 1, 2, 4).reshape(2 * NHLP, D_MODEL))
    bsT = (bs_off.reshape(N_HEADS, N_LEVELS, N_POINTS, 2)
           .transpose(3, 0, 1, 2).reshape(2 * NHLP, 1))
    WaT = Wa.T
    baT = ba.reshape(NHLP, 1)

    val_t = _value_table(input_flatten, Wv, bv)
    idx, w = _prep(q_t, rp_t, WsT, bsT, WaT, baT)

    perm = (0, 4, 2, 3, 1)
    idx_r = (idx.reshape(BS, 4, N_HEADS, N_LEVELS * N_POINTS, NQF)
             .transpose(perm).reshape(R_ROWS * NCON))
    w_r = (w.reshape(BS, 4, N_HEADS, N_LEVELS * N_POINTS, NQF)
           .transpose(perm).reshape(R_ROWS * NCON))
    npad = (R_PAD - R_ROWS) * NCON
    idx_flat = jnp.concatenate([idx_r, jnp.zeros((npad,), jnp.int32)])
    w_flat = jnp.concatenate([w_r, jnp.zeros((npad,), jnp.float32)])

    out_rows = _sc_gather(val_t, idx_flat.reshape(-1, 128), w_flat)
    return _final(out_rows[:R_ROWS].reshape(BS, NQF, D_MODEL), Wo, bo)


def kernel(query, reference_points, input_flatten, input_spatial_shapes,
           Wv, bv, Ws, bs_off, Wa, ba, Wo, bo):
    del input_spatial_shapes
    return _run(query, reference_points, input_flatten, Wv, bv, Ws, bs_off,
                Wa, ba, Wo, bo)


# speedup vs baseline: 2.4537x; 1.0262x over previous
import jax
import jax.numpy as jnp
from jax import lax
from jax.experimental import pallas as pl
from jax.experimental.pallas import tpu as pltpu
from jax.experimental.pallas import tpu_sc as plsc

D_MODEL = 256
N_HEADS = 8
N_LEVELS = 3
N_POINTS = 2
D_HEAD = 32
BS, NQ, FL = 2, 900, 3
H = W = 100
HW = H * W
NQF = NQ * FL
NHLP = N_HEADS * N_LEVELS * N_POINTS
R_ROWS = BS * NQF * N_HEADS
NCON = N_LEVELS * N_POINTS * 4

NW = 32
R_PAD = 45056
ROWS_W = R_PAD // NW
CHUNK = 128
NCHUNK = ROWS_W // CHUNK
IDX_ROWS_W = CHUNK * NCON // 128


def _value_kernel(x_ref, wv_ref, bv_ref, out_ref):
    v = jnp.dot(x_ref[0], wv_ref[...], preferred_element_type=jnp.float32)
    out_ref[0] = v + bv_ref[...][None, :]


def _value_table(input_flatten, Wv, bv):
    thw = 2000
    out = pl.pallas_call(
        _value_kernel,
        grid=(BS, HW // thw),
        in_specs=[
            pl.BlockSpec((1, thw, D_MODEL), lambda b, i: (b, i, 0)),
            pl.BlockSpec((D_MODEL, D_MODEL), lambda b, i: (0, 0)),
            pl.BlockSpec((D_MODEL,), lambda b, i: (0,)),
        ],
        out_specs=pl.BlockSpec((1, thw, D_MODEL), lambda b, i: (b, i, 0)),
        out_shape=jax.ShapeDtypeStruct((BS, HW, D_MODEL), jnp.float32),
    )(input_flatten, Wv, bv)
    return out.reshape(BS * HW * N_HEADS, D_HEAD)


def _prep_kernel(qt_ref, rpt_ref, wst_ref, bst_ref, wat_ref, bat_ref,
                 idx_ref, w_ref):
    b = pl.program_id(0)
    qt = qt_ref[0]
    off = jnp.dot(wst_ref[...], qt, preferred_element_type=jnp.float32)
    off = off + bst_ref[...]
    a = jnp.dot(wat_ref[...], qt, preferred_element_type=jnp.float32)
    a = a + bat_ref[...]
    a3 = a.reshape(N_HEADS, N_LEVELS * N_POINTS, NQF)
    m = jnp.max(a3, axis=1, keepdims=True)
    e = jnp.exp(a3 - m)
    aw = (e / jnp.sum(e, axis=1, keepdims=True)).reshape(NHLP, NQF)

    rpx = rpt_ref[0, 0:1]
    rpy = rpt_ref[0, 1:2]
    x = (rpx + off[0:NHLP] / 100.0) * 100.0 - 0.5
    y = (rpy + off[NHLP:2 * NHLP] / 100.0) * 100.0 - 0.5
    x0 = jnp.floor(x)
    y0 = jnp.floor(y)
    wx1 = x - x0
    wx0 = 1.0 - wx1
    wy1 = y - y0
    wy0 = 1.0 - wy1

    h_arr = lax.broadcasted_iota(jnp.int32, (NHLP, NQF), 0) // (N_LEVELS * N_POINTS)
    for c, (cy, wy, cx, wx) in enumerate(
            ((0, wy0, 0, wx0), (0, wy0, 1, wx1), (1, wy1, 0, wx0), (1, wy1, 1, wx1))):
        yy = y0 + cy
        xx = x0 + cx
        yi = jnp.clip(yy.astype(jnp.int32), 0, H - 1)
        xi = jnp.clip(xx.astype(jnp.int32), 0, W - 1)
        valid = (yy >= 0) & (yy <= float(H - 1)) & (xx >= 0) & (xx <= float(W - 1))
        idx_ref[0, c] = (b * HW + yi * W + xi) * N_HEADS + h_arr
        w_ref[0, c] = wy * wx * valid.astype(jnp.float32) * aw


def _prep(q_t, rp_t, WsT, bsT, WaT, baT):
    idx, w = pl.pallas_call(
        _prep_kernel,
        grid=(BS,),
        in_specs=[
            pl.BlockSpec((1, D_MODEL, NQF), lambda b: (b, 0, 0)),
            pl.BlockSpec((1, 2, NQF), lambda b: (b, 0, 0)),
            pl.BlockSpec((2 * NHLP, D_MODEL), lambda b: (0, 0)),
            pl.BlockSpec((2 * NHLP, 1), lambda b: (0, 0)),
            pl.BlockSpec((NHLP, D_MODEL), lambda b: (0, 0)),
            pl.BlockSpec((NHLP, 1), lambda b: (0, 0)),
        ],
        out_specs=[
            pl.BlockSpec((1, 4, NHLP, NQF), lambda b: (b, 0, 0, 0)),
            pl.BlockSpec((1, 4, NHLP, NQF), lambda b: (b, 0, 0, 0)),
        ],
        out_shape=[
            jax.ShapeDtypeStruct((BS, 4, NHLP, NQF), jnp.int32),
            jax.ShapeDtypeStruct((BS, 4, NHLP, NQF), jnp.float32),
        ],
    )(q_t, rp_t, WsT, bsT, WaT, baT)
    return idx, w


HALF = IDX_ROWS_W // 2
HROWS = CHUNK // 2


def _sc_gather_kernel(val_hbm, idx_hbm, w_hbm, out_hbm, idx_v, w_v, rows_v,
                      out_v, gsemA, gsemB, iwsem):
    wid = lax.axis_index("s") * 2 + lax.axis_index("c")

    def start_iw(c, buf):
        base = pl.multiple_of(wid * ROWS_W + c * CHUNK, 128)
        irow = pl.multiple_of(base * NCON // 128, 8)
        pltpu.async_copy(idx_hbm.at[pl.ds(irow, IDX_ROWS_W)], idx_v.at[buf],
                         iwsem)
        pltpu.async_copy(w_hbm.at[pl.ds(base * NCON, CHUNK * NCON)],
                         w_v.at[buf], iwsem)

    def drain_iw():
        pltpu.make_async_copy(idx_hbm.at[pl.ds(0, IDX_ROWS_W)], idx_v.at[0],
                              iwsem).wait()
        pltpu.make_async_copy(w_hbm.at[pl.ds(0, CHUNK * NCON)], w_v.at[0],
                              iwsem).wait()

    def fire_half(buf, half, sem):
        for j in range(half * HALF, (half + 1) * HALF):
            pltpu.async_copy(val_hbm.at[idx_v.at[buf, j]],
                             rows_v.at[pl.ds(j * 128, 128)], sem)

    def drain_half(buf, half, sem):
        for j in range(half * HALF, (half + 1) * HALF):
            pltpu.make_async_copy(val_hbm.at[idx_v.at[buf, j]],
                                  rows_v.at[pl.ds(j * 128, 128)], sem).wait()

    def accum_half(p, half):
        def row_body(j, _):
            s = j * NCON
            wv0 = w_v[p, pl.ds(s, 16)]
            wv1 = w_v[p, pl.ds(s + 8, 16)]
            acc0 = jnp.zeros((16,), jnp.float32)
            acc1 = jnp.zeros((16,), jnp.float32)
            for k in range(NCON):
                wk = wv0[k] if k < 16 else wv1[k - 8]
                acc0 = acc0 + wk * rows_v[s + k, pl.ds(0, 16)]
                acc1 = acc1 + wk * rows_v[s + k, pl.ds(16, 16)]
            out_v[j, pl.ds(0, 16)] = acc0
            out_v[j, pl.ds(16, 16)] = acc1
            return 0

        lax.fori_loop(half * HROWS, (half + 1) * HROWS, row_body, 0)

    pltpu.sync_copy(idx_hbm.at[pl.ds(pl.multiple_of(wid * ROWS_W * NCON // 128, 8),
                                     IDX_ROWS_W)], idx_v.at[0])
    pltpu.sync_copy(w_hbm.at[pl.ds(wid * ROWS_W * NCON, CHUNK * NCON)],
                    w_v.at[0])
    fire_half(0, 0, gsemA)
    fire_half(0, 1, gsemB)
    start_iw(1, 1)

    def chunk_body(c, _):
        p = jnp.bitwise_and(c, 1)
        q = 1 - p
        base = pl.multiple_of(wid * ROWS_W + c * CHUNK, 128)

        drain_half(p, 0, gsemA)

        @pl.when(c + 1 < NCHUNK)
        def _():
            drain_iw()

        accum_half(p, 0)

        @pl.when(c + 1 < NCHUNK)
        def _():
            fire_half(q, 0, gsemA)

        drain_half(p, 1, gsemB)
        accum_half(p, 1)
        pltpu.sync_copy(out_v, out_hbm.at[pl.ds(base, CHUNK)])

        @pl.when(c + 1 < NCHUNK)
        def _():
            fire_half(q, 1, gsemB)

        @pl.when(c + 2 < NCHUNK)
        def _():
            start_iw(c + 2, p)

        return 0

    lax.fori_loop(0, NCHUNK, chunk_body, 0)


def _sc_gather(val_t, idx_pad2d, w_pad):
    mesh = plsc.VectorSubcoreMesh(core_axis_name="c", subcore_axis_name="s")
    run = pl.kernel(
        _sc_gather_kernel,
        mesh=mesh,
        compiler_params=pltpu.CompilerParams(use_tc_tiling_on_sc=False),
        out_type=jax.ShapeDtypeStruct((R_PAD, D_HEAD), jnp.float32),
        scratch_types=[
            pltpu.VMEM((2, IDX_ROWS_W, 128), jnp.int32),
            pltpu.VMEM((2, CHUNK * NCON), jnp.float32),
            pltpu.VMEM((CHUNK * NCON, D_HEAD), jnp.float32),
            pltpu.VMEM((CHUNK, D_HEAD), jnp.float32),
            pltpu.SemaphoreType.DMA,
            pltpu.SemaphoreType.DMA,
            pltpu.SemaphoreType.DMA,
        ],
    )
    return run(val_t, idx_pad2d, w_pad)


def _final_kernel(s_ref, wo_ref, bo_ref, out_ref):
    out_ref[0] = (jnp.dot(s_ref[0], wo_ref[...], preferred_element_type=jnp.float32)
                  + bo_ref[...][None, :])


def _final(out_rows, Wo, bo):
    out = pl.pallas_call(
        _final_kernel,
        grid=(BS,),
        in_specs=[
            pl.BlockSpec((1, NQF, D_MODEL), lambda b: (b, 0, 0)),
            pl.BlockSpec((D_MODEL, D_MODEL), lambda b: (0, 0)),
            pl.BlockSpec((D_MODEL,), lambda b: (0,)),
        ],
        out_specs=pl.BlockSpec((1, NQF, D_MODEL), lambda b: (b, 0, 0)),
        out_shape=jax.ShapeDtypeStruct((BS, NQF, D_MODEL), jnp.float32),
    )(out_rows, Wo, bo)
    return out.reshape(BS, NQ, FL, D_MODEL)


@jax.jit
def _run(query, reference_points, input_flatten, Wv, bv, Ws, bs_off, Wa, ba,
         Wo, bo):
    q_t = query.reshape(BS, NQF, D_MODEL).transpose(0, 2, 1)
    rp_t = reference_points.reshape(BS, NQF, 2).transpose(0, 2, 1)
    WsT = (Ws.T.reshape(N_HEADS, N_LEVELS, N_POINTS, 2, D_MODEL)
           .transpose(3, 0, 1, 2, 4).reshape(2 * NHLP, D_MODEL))
    bsT = (bs_off.reshape(N_HEADS, N_LEVELS, N_POINTS, 2)
           .transpose(3, 0, 1, 2).reshape(2 * NHLP, 1))
    WaT = Wa.T
    baT = ba.reshape(NHLP, 1)

    val_t = _value_table(input_flatten, Wv, bv)
    idx, w = _prep(q_t, rp_t, WsT, bsT, WaT, baT)

    perm = (0, 4, 2, 3, 1)
    idx_r = (idx.reshape(BS, 4, N_HEADS, N_LEVELS * N_POINTS, NQF)
             .transpose(perm).reshape(R_ROWS * NCON))
    w_r = (w.reshape(BS, 4, N_HEADS, N_LEVELS * N_POINTS, NQF)
           .transpose(perm).reshape(R_ROWS * NCON))
    npad = (R_PAD - R_ROWS) * NCON
    idx_flat = jnp.concatenate([idx_r, jnp.zeros((npad,), jnp.int32)])
    w_flat = jnp.concatenate([w_r, jnp.zeros((npad,), jnp.float32)])

    out_rows = _sc_gather(val_t, idx_flat.reshape(-1, 128), w_flat)
    return _final(out_rows[:R_ROWS].reshape(BS, NQF, D_MODEL), Wo, bo)


def kernel(query, reference_points, input_flatten, input_spatial_shapes,
           Wv, bv, Ws, bs_off, Wa, ba, Wo, bo):
    del input_spatial_shapes
    return _run(query, reference_points, input_flatten, Wv, bv, Ws, bs_off,
                Wa, ba, Wo, bo)
